# baseline (speedup 1.0000x reference)
import functools
import jax
import jax.numpy as jnp
from jax import lax
from jax.experimental import pallas as pl
from jax.experimental.pallas import tpu as pltpu
from jax.experimental.pallas import tpu_sc as plsc

BATCH = 16384
HIST = 50
EMBED = 32
NC, NS = 2, 16
NW = NC * NS
R_PER_W = BATCH // NW
BT_PER_W = R_PER_W // 128

_mesh = plsc.VectorSubcoreMesh(
    core_axis_name="c", subcore_axis_name="s", num_cores=NC, num_subcores=NS
)


@functools.partial(
    pl.kernel,
    out_type=jax.ShapeDtypeStruct((HIST, EMBED // 8, BATCH // 128, 8, 128),
                                  jnp.float32),
    mesh=_mesh,
    scratch_types=[
        pltpu.VMEM((R_PER_W, HIST), jnp.int32),
        pltpu.VMEM((HIST, R_PER_W), jnp.int32),
        pltpu.VMEM((2, BT_PER_W, 128, EMBED), jnp.float32),
        pltpu.VMEM((2, EMBED // 8, 8, 128), jnp.float32),
        pltpu.SemaphoreType.DMA,
        pltpu.SemaphoreType.DMA,
        pltpu.SemaphoreType.DMA,
    ],
    compiler_params=pltpu.CompilerParams(
        use_tc_tiling_on_sc=False, needs_layout_passes=False
    ),
)
def _gather(table_hbm, idx_hbm, out_hbm, idx_v, idx_t, gbuf, tbuf, gsem, w0, w1):
    wsem = (w0, w1)
    wid = lax.axis_index("s") * NC + lax.axis_index("c")
    base = wid * R_PER_W
    lane = lax.iota(jnp.int32, 16)

    pltpu.sync_copy(idx_hbm.at[pl.ds(base, R_PER_W)], idx_v)

    def tr_idx(h, _):
        for k in range(R_PER_W // 16):
            v = plsc.load_gather(idx_v, [lane + (16 * k), jnp.full((16,), h, jnp.int32)])
            idx_t[h, pl.ds(16 * k, 16)] = v
        return ()

    lax.fori_loop(0, HIST, tr_idx, ())

    def start_gathers(h, s):
        for btl in range(BT_PER_W):
            pltpu.async_copy(
                table_hbm.at[idx_t.at[h].at[pl.ds(btl * 128, 128)]],
                gbuf.at[s].at[btl],
                gsem,
            )

    def wait_gathers(h, s):
        for btl in range(BT_PER_W):
            pltpu.make_async_copy(
                table_hbm.at[idx_t.at[h].at[pl.ds(btl * 128, 128)]],
                gbuf.at[s].at[btl],
                gsem,
            ).wait()

    perms = [lax.rem(lane + j, jnp.full((16,), 16, jnp.int32)) for j in range(16)]

    def wait_write(t, h, btl):
        pltpu.make_async_copy(
            tbuf.at[t],
            out_hbm.at[h, :, wid * BT_PER_W + btl],
            wsem[t],
        ).wait()

    def transpose_unit(s, btl, t):
        g = gbuf.at[s].at[btl]
        tb = tbuf.at[t]
        seven = jnp.full((16,), 7, jnp.int32)
        three = jnp.full((16,), 3, jnp.int32)

        def tr_k(k2, _):
            for kk in range(2):
                rowv = lane + (32 * k2 + 16 * kk)
                for c2 in range(EMBED // 16):
                    for j in range(16):
                        colv = perms[j] if c2 == 0 else perms[j] + (16 * c2)
                        v = plsc.load_gather(g, [rowv, colv])
                        plsc.store_scatter(
                            tb,
                            [lax.shift_right_logical(colv, three), lax.bitwise_and(colv, seven), rowv],
                            v,
                        )
            return ()

        lax.fori_loop(0, 4, tr_k, ())

    def start_write(t, h, btl):
        pltpu.async_copy(
            tbuf.at[t],
            out_hbm.at[h, :, wid * BT_PER_W + btl],
            wsem[t],
        )

    def do_h(h, s, h2):
        wait_gathers(h, s)
        for btl in range(BT_PER_W):
            t = btl % 2
            if btl >= 2:
                wait_write(t, h, btl - 2)
            elif s == 1:
                wait_write(t, h - 1, btl + 2)
            else:
                @pl.when(h2 > 0)
                def _():
                    wait_write(t, h - 1, btl + 2)
            transpose_unit(s, btl, t)
            start_write(t, h, btl)

    start_gathers(0, 0)

    def body(h2, _):
        ha = 2 * h2
        start_gathers(ha + 1, 1)
        do_h(ha, 0, h2)

        @pl.when(h2 + 1 < HIST // 2)
        def _():
            start_gathers(ha + 2, 0)

        do_h(ha + 1, 1, h2)
        return ()

    lax.fori_loop(0, HIST // 2, body, ())

    wait_write(0, HIST - 1, 2)
    wait_write(1, HIST - 1, 3)


def kernel(inputs, table):
    op = _gather(table, inputs)
    o5 = jnp.transpose(op, (2, 4, 0, 1, 3))
    return o5.reshape(BATCH, HIST, EMBED)


# speedup vs baseline: 1.1746x; 1.1746x over previous
import functools
import jax
import jax.numpy as jnp
from jax import lax
from jax.experimental import pallas as pl
from jax.experimental.pallas import tpu as pltpu
from jax.experimental.pallas import tpu_sc as plsc

BATCH = 16384
HIST = 50
EMBED = 32
NC, NS = 2, 16
NW = NC * NS
R_PER_W = BATCH // NW
BT_PER_W = R_PER_W // 128

_mesh = plsc.VectorSubcoreMesh(
    core_axis_name="c", subcore_axis_name="s", num_cores=NC, num_subcores=NS
)


@functools.partial(
    pl.kernel,
    out_type=jax.ShapeDtypeStruct((HIST, EMBED // 8, BATCH // 128, 8, 128),
                                  jnp.float32),
    mesh=_mesh,
    scratch_types=[
        pltpu.VMEM((R_PER_W, HIST), jnp.int32),
        pltpu.VMEM((HIST, R_PER_W), jnp.int32),
        pltpu.VMEM((2, BT_PER_W, 128, EMBED), jnp.float32),
        pltpu.VMEM((2, EMBED // 8, 8, 128), jnp.float32),
        pltpu.SemaphoreType.DMA,
        pltpu.SemaphoreType.DMA,
        pltpu.SemaphoreType.DMA,
    ],
    compiler_params=pltpu.CompilerParams(
        use_tc_tiling_on_sc=False, needs_layout_passes=False
    ),
)
def _gather(table_hbm, idx_hbm, out_hbm, idx_v, idx_t, gbuf, tbuf, gsem, w0, w1):
    wsem = (w0, w1)
    wid = lax.axis_index("s") * NC + lax.axis_index("c")
    base = wid * R_PER_W
    lane = lax.iota(jnp.int32, 16)

    pltpu.sync_copy(idx_hbm.at[pl.ds(base, R_PER_W)], idx_v)

    def tr_idx(h, _):
        for k in range(R_PER_W // 16):
            v = plsc.load_gather(idx_v, [lane + (16 * k), jnp.full((16,), h, jnp.int32)])
            idx_t[h, pl.ds(16 * k, 16)] = v
        return ()

    lax.fori_loop(0, HIST, tr_idx, ())

    def start_gathers(h, s):
        for btl in range(BT_PER_W):
            pltpu.async_copy(
                table_hbm.at[idx_t.at[h].at[pl.ds(btl * 128, 128)]],
                gbuf.at[s].at[btl],
                gsem,
            )

    def wait_gathers(h, s):
        for btl in range(BT_PER_W):
            pltpu.make_async_copy(
                table_hbm.at[idx_t.at[h].at[pl.ds(btl * 128, 128)]],
                gbuf.at[s].at[btl],
                gsem,
            ).wait()

    perms = [lax.rem(lane + j, jnp.full((16,), 16, jnp.int32)) for j in range(16)]

    def wait_write(t, h, btl):
        pltpu.make_async_copy(
            tbuf.at[t],
            out_hbm.at[h, :, wid * BT_PER_W + btl],
            wsem[t],
        ).wait()

    def transpose_unit(s, btl, t):
        g = gbuf.at[s].at[btl]
        tb = tbuf.at[t]
        seven = jnp.full((16,), 7, jnp.int32)
        three = jnp.full((16,), 3, jnp.int32)

        @plsc.parallel_loop(0, 8)
        def tr_k(k):
            rowv = lane + 16 * k
            for c2 in range(EMBED // 16):
                for j in range(16):
                    colv = perms[j] if c2 == 0 else perms[j] + (16 * c2)
                    v = plsc.load_gather(g, [rowv, colv])
                    plsc.store_scatter(
                        tb,
                        [lax.shift_right_logical(colv, three), lax.bitwise_and(colv, seven), rowv],
                        v,
                    )

    def start_write(t, h, btl):
        pltpu.async_copy(
            tbuf.at[t],
            out_hbm.at[h, :, wid * BT_PER_W + btl],
            wsem[t],
        )

    def do_h(h, s, h2):
        wait_gathers(h, s)
        for btl in range(BT_PER_W):
            t = btl % 2
            if btl >= 2:
                wait_write(t, h, btl - 2)
            elif s == 1:
                wait_write(t, h - 1, btl + 2)
            else:
                @pl.when(h2 > 0)
                def _():
                    wait_write(t, h - 1, btl + 2)
            transpose_unit(s, btl, t)
            start_write(t, h, btl)

    start_gathers(0, 0)

    def body(h2, _):
        ha = 2 * h2
        start_gathers(ha + 1, 1)
        do_h(ha, 0, h2)

        @pl.when(h2 + 1 < HIST // 2)
        def _():
            start_gathers(ha + 2, 0)

        do_h(ha + 1, 1, h2)
        return ()

    lax.fori_loop(0, HIST // 2, body, ())

    wait_write(0, HIST - 1, 2)
    wait_write(1, HIST - 1, 3)


def kernel(inputs, table):
    op = _gather(table, inputs)
    o5 = jnp.transpose(op, (2, 4, 0, 1, 3))
    return o5.reshape(BATCH, HIST, EMBED)
